# baseline (speedup 1.0000x reference)
import functools

import jax
import jax.numpy as jnp
from jax import lax
from jax.experimental import pallas as pl
from jax.experimental.pallas import tpu as pltpu
from jax.experimental.pallas import tpu_sc as plsc

EMBEDDING_DIM = 64
NUM_EMBEDDINGS = 1024
ROWS_PER_TILE = 512

_NUM_CORES = 2
_NUM_SUBCORES = 16
_NUM_WORKERS = _NUM_CORES * _NUM_SUBCORES
_IDX_CHUNK = 96


def _sum64_tree(parts, axis):
    acc = parts[0]
    for p in parts[1:]:
        acc = acc + p
    a = [lax.index_in_dim(acc, s, axis=axis, keepdims=False) for s in range(8)]
    return ((a[0] + a[4]) + (a[2] + a[6])) + ((a[1] + a[5]) + (a[3] + a[7]))


def _argmin_body(x_ref, w_ref, idx_ref):
    xb = x_ref[...]
    w = w_ref[...]
    xw = jnp.dot(xb, w)
    x2 = xb * xb
    xsq = _sum64_tree([x2[:, 8 * r:8 * r + 8] for r in range(8)], axis=1)
    w2 = w * w
    wsq = _sum64_tree([w2[8 * r:8 * r + 8, :] for r in range(8)], axis=0)
    distances = xsq[:, None] - 2.0 * xw + wsq[None, :]
    idx = jnp.argmin(distances, axis=1).astype(jnp.int32)
    idx_ref[...] = jnp.reshape(idx, (1, 1, ROWS_PER_TILE))


def _encode_indices(flat, w):
    n = flat.shape[0]
    grid = n // ROWS_PER_TILE
    idx = pl.pallas_call(
        _argmin_body,
        grid=(grid,),
        in_specs=[
            pl.BlockSpec((ROWS_PER_TILE, EMBEDDING_DIM), lambda i: (i, 0)),
            pl.BlockSpec((EMBEDDING_DIM, NUM_EMBEDDINGS), lambda i: (0, 0)),
        ],
        out_specs=pl.BlockSpec((1, 1, ROWS_PER_TILE), lambda i: (i, 0, 0)),
        out_shape=jax.ShapeDtypeStruct((grid, 1, ROWS_PER_TILE), jnp.int32),
        compiler_params=pltpu.CompilerParams(
            dimension_semantics=("parallel",)),
    )(flat, w)
    return jnp.reshape(idx, (n,))


def _make_sc_gather(n):
    b_per_w = n // _NUM_WORKERS
    n_chunks = b_per_w // _IDX_CHUNK
    mesh = plsc.VectorSubcoreMesh(core_axis_name="c", subcore_axis_name="s")

    @functools.partial(
        pl.kernel, mesh=mesh,
        out_type=jax.ShapeDtypeStruct((n, EMBEDDING_DIM), jnp.float32),
        scratch_types=[
            pltpu.VMEM((b_per_w,), jnp.int32),
            pltpu.VMEM((b_per_w, EMBEDDING_DIM), jnp.float32),
            pltpu.SemaphoreType.DMA,
        ],
        compiler_params=pltpu.CompilerParams(use_tc_tiling_on_sc=False),
    )
    def gather_kernel(table_hbm, idx_hbm, out_hbm, idx_v, rows_v, sem):
        wid = lax.axis_index("s") * _NUM_CORES + lax.axis_index("c")
        base = wid * b_per_w
        pltpu.sync_copy(idx_hbm.at[pl.ds(base, b_per_w)], idx_v)
        copies = []
        for j in range(n_chunks):
            off = j * _IDX_CHUNK
            copies.append(pltpu.async_copy(
                table_hbm.at[idx_v.at[pl.ds(off, _IDX_CHUNK)]],
                rows_v.at[pl.ds(off, _IDX_CHUNK)],
                sem))
        for c in copies:
            c.wait()
        pltpu.sync_copy(rows_v, out_hbm.at[pl.ds(base, b_per_w)])

    return gather_kernel


def kernel(x, w):
    flat = jnp.reshape(x, (-1, EMBEDDING_DIM))
    n = flat.shape[0]
    indices = _encode_indices(flat, w)
    table = jnp.transpose(w)
    quantized = _make_sc_gather(n)(table, indices)
    return jnp.reshape(quantized, x.shape)


# speedup vs baseline: 1.5124x; 1.5124x over previous
import functools

import jax
import jax.numpy as jnp
from jax import lax
from jax.experimental import pallas as pl
from jax.experimental.pallas import tpu as pltpu
from jax.experimental.pallas import tpu_sc as plsc

EMBEDDING_DIM = 64
NUM_EMBEDDINGS = 1024
ROWS_PER_TILE = 512


def _vq_body(x_ref, w_ref, xsq_ref, wsq_ref, o_ref):
    xb = x_ref[...]
    w = w_ref[...]
    xw = jnp.dot(xb, w)
    distances = xsq_ref[...] - 2.0 * xw + wsq_ref[...]
    idx = jnp.argmin(distances, axis=1)
    cols = lax.broadcasted_iota(jnp.int32, distances.shape, 1)
    onehot = (idx[:, None] == cols).astype(jnp.float32)
    o_ref[...] = lax.dot_general(
        onehot, w, (((1,), (1,)), ((), ())),
        preferred_element_type=jnp.float32)


def kernel(x, w):
    flat = jnp.reshape(x, (-1, EMBEDDING_DIM))
    n = flat.shape[0]
    grid = n // ROWS_PER_TILE
    xsq = jnp.sum(flat ** 2, axis=1, keepdims=True)
    wsq = jnp.sum(w ** 2, axis=0, keepdims=True)
    out = pl.pallas_call(
        _vq_body,
        grid=(grid,),
        in_specs=[
            pl.BlockSpec((ROWS_PER_TILE, EMBEDDING_DIM), lambda i: (i, 0)),
            pl.BlockSpec((EMBEDDING_DIM, NUM_EMBEDDINGS), lambda i: (0, 0)),
            pl.BlockSpec((ROWS_PER_TILE, 1), lambda i: (i, 0)),
            pl.BlockSpec((1, NUM_EMBEDDINGS), lambda i: (0, 0)),
        ],
        out_specs=pl.BlockSpec((ROWS_PER_TILE, EMBEDDING_DIM), lambda i: (i, 0)),
        out_shape=jax.ShapeDtypeStruct((n, EMBEDDING_DIM), jnp.float32),
        compiler_params=pltpu.CompilerParams(
            dimension_semantics=("parallel",)),
    )(flat, w, xsq, wsq)
    return jnp.reshape(out, x.shape)


# speedup vs baseline: 1.5409x; 1.0189x over previous
import functools

import jax
import jax.numpy as jnp
from jax import lax
from jax.experimental import pallas as pl
from jax.experimental.pallas import tpu as pltpu
from jax.experimental.pallas import tpu_sc as plsc

EMBEDDING_DIM = 64
NUM_EMBEDDINGS = 1024
ROWS_PER_TILE = 512


def _vq_body(x_ref, w_ref, xsq_ref, wsq_ref, o_ref):
    xb = x_ref[...]
    w = w_ref[...]
    xw = jnp.dot(xb, w)
    distances = xsq_ref[...] - 2.0 * xw + wsq_ref[...]
    idx = jnp.argmin(distances, axis=1)
    cols = lax.broadcasted_iota(jnp.int32, distances.shape, 1)
    onehot = (idx[:, None] == cols).astype(jnp.bfloat16)
    o_ref[...] = lax.dot_general(
        onehot, w.astype(jnp.bfloat16), (((1,), (1,)), ((), ())),
        preferred_element_type=jnp.float32)


def kernel(x, w):
    flat = jnp.reshape(x, (-1, EMBEDDING_DIM))
    n = flat.shape[0]
    grid = n // ROWS_PER_TILE
    xsq = jnp.sum(flat ** 2, axis=1, keepdims=True)
    wsq = jnp.sum(w ** 2, axis=0, keepdims=True)
    out = pl.pallas_call(
        _vq_body,
        grid=(grid,),
        in_specs=[
            pl.BlockSpec((ROWS_PER_TILE, EMBEDDING_DIM), lambda i: (i, 0)),
            pl.BlockSpec((EMBEDDING_DIM, NUM_EMBEDDINGS), lambda i: (0, 0)),
            pl.BlockSpec((ROWS_PER_TILE, 1), lambda i: (i, 0)),
            pl.BlockSpec((1, NUM_EMBEDDINGS), lambda i: (0, 0)),
        ],
        out_specs=pl.BlockSpec((ROWS_PER_TILE, EMBEDDING_DIM), lambda i: (i, 0)),
        out_shape=jax.ShapeDtypeStruct((n, EMBEDDING_DIM), jnp.float32),
        compiler_params=pltpu.CompilerParams(
            dimension_semantics=("parallel",)),
    )(flat, w, xsq, wsq)
    return jnp.reshape(out, x.shape)
